# baseline (speedup 1.0000x reference)
import functools

import jax
import jax.numpy as jnp
from jax import lax
from jax.experimental import pallas as pl
from jax.experimental.pallas import tpu as pltpu
from jax.experimental.pallas import tpu_sc as plsc

_B = 16384
_FE = 26
_FN = 13
_V = 100000
_VP2 = 100096
_VP1 = 102400
_E = 16
_EPS = 1e-5
_SE = _FE * _E
_SN = _FN * _E

_NW = 32
_BC = _B // _NW


def _build_sc_gather():
    mesh = plsc.VectorSubcoreMesh(core_axis_name="c", subcore_axis_name="s")

    @functools.partial(
        pl.kernel,
        mesh=mesh,
        compiler_params=pltpu.CompilerParams(use_tc_tiling_on_sc=False),
        out_type=(
            jax.ShapeDtypeStruct((_FE, _B), jnp.float32),
            jax.ShapeDtypeStruct((_SE, _B), jnp.float32),
        ),
        scratch_types=[
            pltpu.VMEM((_BC,), jnp.int32),
            pltpu.VMEM((4, 128), jnp.int32),
            pltpu.VMEM((_E, _BC), jnp.int32),
            pltpu.VMEM((_BC,), jnp.float32),
            pltpu.VMEM((_E, _BC), jnp.float32),
            pltpu.SemaphoreType.DMA,
            pltpu.SemaphoreType.DMA,
        ],
    )
    def sc_gather(idx_hbm, first_hbm, second_hbm, fe_out, se_out,
                  vchunk, idxf, idxs, fvals, svals, sem_f, sem_s):
        wid = lax.axis_index("s") * 2 + lax.axis_index("c")
        b0 = pl.multiple_of(wid * _BC, _BC)

        def per_field(f, carry):
            pltpu.sync_copy(idx_hbm.at[f, pl.ds(b0, _BC)], vchunk)

            def build_first(t, c):
                v16 = vchunk[pl.ds(t * 16, 16)]
                idxf[t // 8, pl.ds((t % 8) * 16, 16)] = v16 + f * _VP1
                return c
            lax.fori_loop(0, _BC // 16, build_first, 0)

            def build_second(e, c):
                base = (f * _E + e) * _VP2

                def inner(t, c2):
                    v16 = vchunk[pl.ds(t * 16, 16)]
                    idxs[e, pl.ds(t * 16, 16)] = v16 + base
                    return c2
                lax.fori_loop(0, _BC // 16, inner, 0)
                return c
            lax.fori_loop(0, _E, build_second, 0)

            waits = []
            for q in range(4):
                waits.append(pltpu.async_copy(
                    first_hbm.at[idxf.at[q]],
                    fvals.at[pl.ds(q * 128, 128)], sem_f))
            for e in range(_E):
                waits.append(pltpu.async_copy(
                    second_hbm.at[idxs.at[e]],
                    svals.at[e], sem_s))
            for w in waits:
                w.wait()

            pltpu.sync_copy(fvals, fe_out.at[f, pl.ds(b0, _BC)])
            pltpu.sync_copy(
                svals, se_out.at[pl.ds(f * _E, _E), pl.ds(b0, _BC)])
            return carry

        lax.fori_loop(0, _FE, per_field, 0)

    return sc_gather


_LB = 2048
_NLB = _B // _LB


def _reduce_body(num_ref, fe_ref, se_ref,
                 o_sn, o_qn, o_sf, o_qf, o_ss, o_qs):
    @pl.when(pl.program_id(0) == 0)
    def _init():
        o_sn[...] = jnp.zeros_like(o_sn)
        o_qn[...] = jnp.zeros_like(o_qn)
        o_sf[...] = jnp.zeros_like(o_sf)
        o_qf[...] = jnp.zeros_like(o_qf)
        o_ss[...] = jnp.zeros_like(o_ss)
        o_qs[...] = jnp.zeros_like(o_qs)

    n = num_ref[...]
    f = fe_ref[...]
    s = se_ref[...]
    o_sn[...] += jnp.sum(n, axis=1, keepdims=True)
    o_qn[...] += jnp.sum(n * n, axis=1, keepdims=True)
    o_sf[...] += jnp.sum(f, axis=1, keepdims=True)
    o_qf[...] += jnp.sum(f * f, axis=1, keepdims=True)
    o_ss[...] += jnp.sum(s, axis=1, keepdims=True)
    o_qs[...] += jnp.sum(s * s, axis=1, keepdims=True)


def _norm_body(num_ref, fe_ref, se_ref,
               s_sn, s_qn, s_sf, s_qf, s_ss, s_qs,
               ng, nbt, w1, fg_e, fb_e, fg_n, fb_n,
               sg_e, sb_e, sg_n, sb_n, w2m, w2qm,
               w2selt, ext, grt, e13t,
               fo, so):
    hi = jax.lax.Precision.HIGHEST
    bf = float(_B)

    mn = s_sn[...] * (1.0 / bf)
    vn = s_qn[...] * (1.0 / bf) - mn * mn
    isn = lax.rsqrt(vn + _EPS)
    nsc = ng[...] * isn
    nsh = nbt[...] - mn * nsc
    num_bn = num_ref[...] * nsc + nsh

    mf = s_sf[...] * (1.0 / bf)
    vf = s_qf[...] * (1.0 / bf) - mf * mf
    fsc = fg_e[...] * lax.rsqrt(vf + _EPS)
    fo_e = fe_ref[...] * fsc + (fb_e[...] - mf * fsc)

    mu_nb = nbt[...]
    var_nb = ng[...] * ng[...] * vn / (vn + _EPS)
    m1n = w1[...] * mu_nb
    v1n = w1[...] * w1[...] * var_nb
    sc1n = fg_n[...] * lax.rsqrt(v1n + _EPS)
    fo_n = num_bn * (w1[...] * sc1n) + (fb_n[...] - m1n * sc1n)
    fo[...] = jnp.concatenate([fo_e, fo_n], axis=0)

    be = float(_B * _E)
    m2 = jnp.dot(grt[...], s_ss[...], precision=hi) * (1.0 / be)
    v2 = jnp.dot(grt[...], s_qs[...], precision=hi) * (1.0 / be) - m2 * m2
    sc2 = sg_e[...] * lax.rsqrt(v2 + _EPS)
    sh2 = sb_e[...] - m2 * sc2
    so_e = (se_ref[...] * jnp.dot(ext[...], sc2, precision=hi)
            + jnp.dot(ext[...], sh2, precision=hi))

    e_nb2 = var_nb + mu_nb * mu_nb
    m2n = mu_nb * w2m[...]
    v2n = e_nb2 * w2qm[...] - m2n * m2n
    sc2n = sg_n[...] * lax.rsqrt(v2n + _EPS)
    sh2n = sb_n[...] - m2n * sc2n
    so_n = (jnp.dot(w2selt[...] * jnp.dot(e13t[...], sc2n, precision=hi),
                    num_bn, precision=hi)
            + jnp.dot(e13t[...], sh2n, precision=hi))
    so[...] = jnp.concatenate([so_e, so_n], axis=0)


def _col(r):
    return pl.BlockSpec((r, 1), lambda i: (0, 0))


def _blk(r):
    return pl.BlockSpec((r, _LB), lambda i: (0, i))


def _full(r, c):
    return pl.BlockSpec((r, c), lambda i: (0, 0))


def _gather_call(idx_t, first_flat, second_flat):
    return _build_sc_gather()(idx_t, first_flat, second_flat)


def _f32(shape):
    return jax.ShapeDtypeStruct(shape, jnp.float32)


def kernel(emb_indices, numeric, first_table, second_table, first_num_w,
           second_num_w, numeric_gamma, numeric_beta, first_gamma,
           first_beta, second_gamma, second_beta):
    idx_t = emb_indices.T
    num_t = numeric.T
    ftp = jnp.pad(first_table, ((0, 0), (0, _VP1 - _V)))
    ft128 = jax.lax.optimization_barrier(
        ftp.reshape(_FE * _VP1 // 128, 128))
    first_flat = ft128.reshape(_FE * _VP1)
    spad = jnp.pad(second_table, ((0, 0), (0, _VP2 - _V), (0, 0)))
    st128 = jax.lax.optimization_barrier(
        jnp.swapaxes(spad, 1, 2).reshape(_FE * _E * _VP2 // 128, 128))
    second_flat = st128.reshape(_FE * _E * _VP2)

    fe_t, se_t = _gather_call(idx_t, first_flat, second_flat)

    sums = pl.pallas_call(
        _reduce_body,
        grid=(_NLB,),
        in_specs=[_blk(_FN), _blk(_FE), _blk(_SE)],
        out_specs=[_col(_FN), _col(_FN), _col(_FE), _col(_FE),
                   _col(_SE), _col(_SE)],
        out_shape=[_f32((_FN, 1)), _f32((_FN, 1)), _f32((_FE, 1)),
                   _f32((_FE, 1)), _f32((_SE, 1)), _f32((_SE, 1))],
    )(num_t, fe_t, se_t)

    c208 = jnp.arange(_SN, dtype=jnp.int32)
    j13 = jnp.arange(_FN, dtype=jnp.int32)
    e13t = (c208[:, None] // _E == j13[None, :]).astype(jnp.float32)
    w2col = second_num_w.reshape(_SN, 1)
    w2selt = e13t * w2col
    c416 = jnp.arange(_SE, dtype=jnp.int32)
    f26 = jnp.arange(_FE, dtype=jnp.int32)
    ext = (c416[:, None] // _E == f26[None, :]).astype(jnp.float32)
    grt = ext.T
    w2m = jnp.mean(second_num_w, axis=1)[:, None]
    w2qm = jnp.mean(second_num_w * second_num_w, axis=1)[:, None]

    params = (
        numeric_gamma[:, None], numeric_beta[:, None], first_num_w[:, None],
        first_gamma[:_FE, None], first_beta[:_FE, None],
        first_gamma[_FE:, None], first_beta[_FE:, None],
        second_gamma[:_FE, None], second_beta[:_FE, None],
        second_gamma[_FE:, None], second_beta[_FE:, None],
        w2m, w2qm, w2selt, ext, grt, e13t,
    )
    param_specs = [
        _col(_FN), _col(_FN), _col(_FN),
        _col(_FE), _col(_FE), _col(_FN), _col(_FN),
        _col(_FE), _col(_FE), _col(_FN), _col(_FN),
        _col(_FN), _col(_FN), _full(_SN, _FN), _full(_SE, _FE),
        _full(_FE, _SE), _full(_SN, _FN),
    ]

    fo, so = pl.pallas_call(
        _norm_body,
        grid=(_NLB,),
        in_specs=[_blk(_FN), _blk(_FE), _blk(_SE)]
                 + [_col(_FN), _col(_FN), _col(_FE), _col(_FE),
                    _col(_SE), _col(_SE)]
                 + param_specs,
        out_specs=[_blk(_FE + _FN), _blk(_SE + _SN)],
        out_shape=[_f32((_FE + _FN, _B)), _f32((_SE + _SN, _B))],
    )(num_t, fe_t, se_t, *sums, *params)

    first_out = fo.T
    second_out = so.reshape(_FE + _FN, _E, _B).transpose(2, 0, 1)
    return first_out, second_out


# speedup vs baseline: 1.0554x; 1.0554x over previous
import functools

import jax
import jax.numpy as jnp
from jax import lax
from jax.experimental import pallas as pl
from jax.experimental.pallas import tpu as pltpu
from jax.experimental.pallas import tpu_sc as plsc

_B = 16384
_FE = 26
_FN = 13
_V = 100000
_VP2 = 100096
_VP1 = 102400
_E = 16
_EPS = 1e-5
_SE = _FE * _E
_SN = _FN * _E

_NW = 32
_BC = _B // _NW


def _build_sc_gather():
    mesh = plsc.VectorSubcoreMesh(core_axis_name="c", subcore_axis_name="s")

    @functools.partial(
        pl.kernel,
        mesh=mesh,
        compiler_params=pltpu.CompilerParams(use_tc_tiling_on_sc=False),
        out_type=(
            jax.ShapeDtypeStruct((_FE, _B), jnp.float32),
            jax.ShapeDtypeStruct((_SE, _B), jnp.float32),
        ),
        scratch_types=[
            pltpu.VMEM((2, _BC), jnp.int32),
            pltpu.VMEM((2, _E, _BC), jnp.int32),
            pltpu.VMEM((2, _BC), jnp.float32),
            pltpu.VMEM((2, _E, _BC), jnp.float32),
            pltpu.SemaphoreType.DMA,
            pltpu.SemaphoreType.DMA,
            pltpu.SemaphoreType.DMA,
        ],
    )
    def sc_gather(idx1_hbm, idx2_hbm, first_hbm, second_hbm, fe_out, se_out,
                  idxf, idxs, fvals, svals, sem_f, sem_s, sem_w):
        wid = lax.axis_index("s") * 2 + lax.axis_index("c")
        b0 = pl.multiple_of(wid * _BC, _BC)

        def load(f, par):
            pltpu.sync_copy(idx1_hbm.at[f, pl.ds(b0, _BC)], idxf.at[par])
            pltpu.sync_copy(idx2_hbm.at[pl.ds(f * _E, _E), pl.ds(b0, _BC)],
                            idxs.at[par])

        wb = {0: [], 1: []}
        load(0, 0)
        for f in range(_FE):
            par, nxt = f % 2, (f + 1) % 2
            for h in wb[par]:
                h.wait()
            wb[par] = []
            gw = [pltpu.async_copy(first_hbm.at[idxf.at[par]],
                                   fvals.at[par], sem_f)]
            for e in range(_E):
                gw.append(pltpu.async_copy(
                    second_hbm.at[idxs.at[par, e]],
                    svals.at[par, e], sem_s))
            if f + 1 < _FE:
                load(f + 1, nxt)
            for h in gw:
                h.wait()
            wb[par] = [
                pltpu.async_copy(fvals.at[par],
                                 fe_out.at[f, pl.ds(b0, _BC)], sem_w),
                pltpu.async_copy(
                    svals.at[par],
                    se_out.at[pl.ds(f * _E, _E), pl.ds(b0, _BC)], sem_w),
            ]
        for par in (0, 1):
            for h in wb[par]:
                h.wait()

    return sc_gather


_LB = 2048
_NLB = _B // _LB


def _reduce_body(num_ref, fe_ref, se_ref,
                 o_sn, o_qn, o_sf, o_qf, o_ss, o_qs):
    @pl.when(pl.program_id(0) == 0)
    def _init():
        o_sn[...] = jnp.zeros_like(o_sn)
        o_qn[...] = jnp.zeros_like(o_qn)
        o_sf[...] = jnp.zeros_like(o_sf)
        o_qf[...] = jnp.zeros_like(o_qf)
        o_ss[...] = jnp.zeros_like(o_ss)
        o_qs[...] = jnp.zeros_like(o_qs)

    n = num_ref[...]
    f = fe_ref[...]
    s = se_ref[...]
    o_sn[...] += jnp.sum(n, axis=1, keepdims=True)
    o_qn[...] += jnp.sum(n * n, axis=1, keepdims=True)
    o_sf[...] += jnp.sum(f, axis=1, keepdims=True)
    o_qf[...] += jnp.sum(f * f, axis=1, keepdims=True)
    o_ss[...] += jnp.sum(s, axis=1, keepdims=True)
    o_qs[...] += jnp.sum(s * s, axis=1, keepdims=True)


def _norm_body(num_ref, fe_ref, se_ref,
               s_sn, s_qn, s_sf, s_qf, s_ss, s_qs,
               ng, nbt, w1, fg_e, fb_e, fg_n, fb_n,
               sg_e, sb_e, sg_n, sb_n, w2m, w2qm,
               w2selt, ext, grt, e13t,
               fo, so):
    hi = jax.lax.Precision.HIGHEST
    bf = float(_B)

    mn = s_sn[...] * (1.0 / bf)
    vn = s_qn[...] * (1.0 / bf) - mn * mn
    isn = lax.rsqrt(vn + _EPS)
    nsc = ng[...] * isn
    nsh = nbt[...] - mn * nsc
    num_bn = num_ref[...] * nsc + nsh

    mf = s_sf[...] * (1.0 / bf)
    vf = s_qf[...] * (1.0 / bf) - mf * mf
    fsc = fg_e[...] * lax.rsqrt(vf + _EPS)
    fo_e = fe_ref[...] * fsc + (fb_e[...] - mf * fsc)

    mu_nb = nbt[...]
    var_nb = ng[...] * ng[...] * vn / (vn + _EPS)
    m1n = w1[...] * mu_nb
    v1n = w1[...] * w1[...] * var_nb
    sc1n = fg_n[...] * lax.rsqrt(v1n + _EPS)
    fo_n = num_bn * (w1[...] * sc1n) + (fb_n[...] - m1n * sc1n)
    fo[...] = jnp.concatenate([fo_e, fo_n], axis=0)

    be = float(_B * _E)
    m2 = jnp.dot(grt[...], s_ss[...], precision=hi) * (1.0 / be)
    v2 = jnp.dot(grt[...], s_qs[...], precision=hi) * (1.0 / be) - m2 * m2
    sc2 = sg_e[...] * lax.rsqrt(v2 + _EPS)
    sh2 = sb_e[...] - m2 * sc2
    so_e = (se_ref[...] * jnp.dot(ext[...], sc2, precision=hi)
            + jnp.dot(ext[...], sh2, precision=hi))

    e_nb2 = var_nb + mu_nb * mu_nb
    m2n = mu_nb * w2m[...]
    v2n = e_nb2 * w2qm[...] - m2n * m2n
    sc2n = sg_n[...] * lax.rsqrt(v2n + _EPS)
    sh2n = sb_n[...] - m2n * sc2n
    so_n = (jnp.dot(w2selt[...] * jnp.dot(e13t[...], sc2n, precision=hi),
                    num_bn, precision=hi)
            + jnp.dot(e13t[...], sh2n, precision=hi))
    so[...] = jnp.concatenate([so_e, so_n], axis=0)


def _col(r):
    return pl.BlockSpec((r, 1), lambda i: (0, 0))


def _blk(r):
    return pl.BlockSpec((r, _LB), lambda i: (0, i))


def _full(r, c):
    return pl.BlockSpec((r, c), lambda i: (0, 0))


def _gather_call(idx1, idx2, first_flat, second_flat):
    return _build_sc_gather()(idx1, idx2, first_flat, second_flat)


def _f32(shape):
    return jax.ShapeDtypeStruct(shape, jnp.float32)


def kernel(emb_indices, numeric, first_table, second_table, first_num_w,
           second_num_w, numeric_gamma, numeric_beta, first_gamma,
           first_beta, second_gamma, second_beta):
    idx_t = emb_indices.T
    num_t = numeric.T
    ftp = jnp.pad(first_table, ((0, 0), (0, _VP1 - _V)))
    ft128 = jax.lax.optimization_barrier(
        ftp.reshape(_FE * _VP1 // 128, 128))
    first_flat = ft128.reshape(_FE * _VP1)
    spad = jnp.pad(second_table, ((0, 0), (0, _VP2 - _V), (0, 0)))
    st128 = jax.lax.optimization_barrier(
        jnp.swapaxes(spad, 1, 2).reshape(_FE * _E * _VP2 // 128, 128))
    second_flat = st128.reshape(_FE * _E * _VP2)

    idx1 = idx_t + (jnp.arange(_FE, dtype=jnp.int32) * _VP1)[:, None]
    idx2 = (jnp.repeat(idx_t, _E, axis=0)
            + (jnp.arange(_SE, dtype=jnp.int32) * _VP2)[:, None])
    fe_t, se_t = _gather_call(idx1, idx2, first_flat, second_flat)

    sums = pl.pallas_call(
        _reduce_body,
        grid=(_NLB,),
        in_specs=[_blk(_FN), _blk(_FE), _blk(_SE)],
        out_specs=[_col(_FN), _col(_FN), _col(_FE), _col(_FE),
                   _col(_SE), _col(_SE)],
        out_shape=[_f32((_FN, 1)), _f32((_FN, 1)), _f32((_FE, 1)),
                   _f32((_FE, 1)), _f32((_SE, 1)), _f32((_SE, 1))],
    )(num_t, fe_t, se_t)

    c208 = jnp.arange(_SN, dtype=jnp.int32)
    j13 = jnp.arange(_FN, dtype=jnp.int32)
    e13t = (c208[:, None] // _E == j13[None, :]).astype(jnp.float32)
    w2col = second_num_w.reshape(_SN, 1)
    w2selt = e13t * w2col
    c416 = jnp.arange(_SE, dtype=jnp.int32)
    f26 = jnp.arange(_FE, dtype=jnp.int32)
    ext = (c416[:, None] // _E == f26[None, :]).astype(jnp.float32)
    grt = ext.T
    w2m = jnp.mean(second_num_w, axis=1)[:, None]
    w2qm = jnp.mean(second_num_w * second_num_w, axis=1)[:, None]

    params = (
        numeric_gamma[:, None], numeric_beta[:, None], first_num_w[:, None],
        first_gamma[:_FE, None], first_beta[:_FE, None],
        first_gamma[_FE:, None], first_beta[_FE:, None],
        second_gamma[:_FE, None], second_beta[:_FE, None],
        second_gamma[_FE:, None], second_beta[_FE:, None],
        w2m, w2qm, w2selt, ext, grt, e13t,
    )
    param_specs = [
        _col(_FN), _col(_FN), _col(_FN),
        _col(_FE), _col(_FE), _col(_FN), _col(_FN),
        _col(_FE), _col(_FE), _col(_FN), _col(_FN),
        _col(_FN), _col(_FN), _full(_SN, _FN), _full(_SE, _FE),
        _full(_FE, _SE), _full(_SN, _FN),
    ]

    fo, so = pl.pallas_call(
        _norm_body,
        grid=(_NLB,),
        in_specs=[_blk(_FN), _blk(_FE), _blk(_SE)]
                 + [_col(_FN), _col(_FN), _col(_FE), _col(_FE),
                    _col(_SE), _col(_SE)]
                 + param_specs,
        out_specs=[_blk(_FE + _FN), _blk(_SE + _SN)],
        out_shape=[_f32((_FE + _FN, _B)), _f32((_SE + _SN, _B))],
    )(num_t, fe_t, se_t, *sums, *params)

    first_out = fo.T
    second_out = so.reshape(_FE + _FN, _E, _B).transpose(2, 0, 1)
    return first_out, second_out


# speedup vs baseline: 1.0733x; 1.0170x over previous
import functools

import jax
import jax.numpy as jnp
from jax import lax
from jax.experimental import pallas as pl
from jax.experimental.pallas import tpu as pltpu
from jax.experimental.pallas import tpu_sc as plsc

_B = 16384
_FE = 26
_FN = 13
_V = 100000
_VP2 = 100096
_VP1 = 102400
_E = 16
_EPS = 1e-5
_SE = _FE * _E
_SN = _FN * _E

_NW = 32
_BC = _B // _NW


def _build_sc_gather():
    mesh = plsc.VectorSubcoreMesh(core_axis_name="c", subcore_axis_name="s")

    @functools.partial(
        pl.kernel,
        mesh=mesh,
        compiler_params=pltpu.CompilerParams(use_tc_tiling_on_sc=False),
        out_type=(
            jax.ShapeDtypeStruct((_FE, _B), jnp.float32),
            jax.ShapeDtypeStruct((_SE, _B), jnp.float32),
        ),
        scratch_types=[
            pltpu.VMEM((2, _BC), jnp.int32),
            pltpu.VMEM((2, _E, _BC), jnp.int32),
            pltpu.VMEM((2, _BC), jnp.float32),
            pltpu.VMEM((2, _E, _BC), jnp.float32),
            pltpu.SemaphoreType.DMA,
            pltpu.SemaphoreType.DMA,
            pltpu.SemaphoreType.DMA,
            pltpu.SemaphoreType.DMA,
        ],
    )
    def sc_gather(idx1_hbm, idx2_hbm, first_hbm, second_hbm, fe_out, se_out,
                  idxf, idxs, fvals, svals, sem_g0, sem_g1, sem_w0, sem_w1):
        wid = lax.axis_index("s") * 2 + lax.axis_index("c")
        b0 = pl.multiple_of(wid * _BC, _BC)
        sem_g = (sem_g0, sem_g1)
        sem_w = (sem_w0, sem_w1)

        def load(f, par):
            pltpu.sync_copy(idx1_hbm.at[f, pl.ds(b0, _BC)], idxf.at[par])
            pltpu.sync_copy(idx2_hbm.at[pl.ds(f * _E, _E), pl.ds(b0, _BC)],
                            idxs.at[par])

        def fire(f, par):
            gw = [pltpu.async_copy(first_hbm.at[idxf.at[par]],
                                   fvals.at[par], sem_g[par])]
            for e in range(_E):
                gw.append(pltpu.async_copy(
                    second_hbm.at[idxs.at[par, e]],
                    svals.at[par, e], sem_g[par]))
            return gw

        def writeback(f, par):
            return [
                pltpu.async_copy(fvals.at[par],
                                 fe_out.at[f, pl.ds(b0, _BC)], sem_w[par]),
                pltpu.async_copy(
                    svals.at[par],
                    se_out.at[pl.ds(f * _E, _E), pl.ds(b0, _BC)],
                    sem_w[par]),
            ]

        wb = {0: [], 1: []}
        gw = {0: [], 1: []}
        load(0, 0)
        gw[0] = fire(0, 0)
        load(1, 1)
        for f in range(1, _FE + 1):
            par, prv = f % 2, (f - 1) % 2
            if f < _FE:
                for h in wb[par]:
                    h.wait()
                wb[par] = []
                gw[par] = fire(f, par)
            for h in gw[prv]:
                h.wait()
            gw[prv] = []
            wb[prv] = writeback(f - 1, prv)
            if f + 1 < _FE:
                load(f + 1, prv)
        for par in (0, 1):
            for h in wb[par]:
                h.wait()

    return sc_gather


_LB = 2048
_NLB = _B // _LB


def _reduce_body(num_ref, fe_ref, se_ref,
                 o_sn, o_qn, o_sf, o_qf, o_ss, o_qs):
    @pl.when(pl.program_id(0) == 0)
    def _init():
        o_sn[...] = jnp.zeros_like(o_sn)
        o_qn[...] = jnp.zeros_like(o_qn)
        o_sf[...] = jnp.zeros_like(o_sf)
        o_qf[...] = jnp.zeros_like(o_qf)
        o_ss[...] = jnp.zeros_like(o_ss)
        o_qs[...] = jnp.zeros_like(o_qs)

    n = num_ref[...]
    f = fe_ref[...]
    s = se_ref[...]
    o_sn[...] += jnp.sum(n, axis=1, keepdims=True)
    o_qn[...] += jnp.sum(n * n, axis=1, keepdims=True)
    o_sf[...] += jnp.sum(f, axis=1, keepdims=True)
    o_qf[...] += jnp.sum(f * f, axis=1, keepdims=True)
    o_ss[...] += jnp.sum(s, axis=1, keepdims=True)
    o_qs[...] += jnp.sum(s * s, axis=1, keepdims=True)


def _norm_body(num_ref, fe_ref, se_ref,
               s_sn, s_qn, s_sf, s_qf, s_ss, s_qs,
               ng, nbt, w1, fg_e, fb_e, fg_n, fb_n,
               sg_e, sb_e, sg_n, sb_n, w2m, w2qm,
               w2selt, ext, grt, e13t,
               fo, so):
    hi = jax.lax.Precision.HIGHEST
    bf = float(_B)

    mn = s_sn[...] * (1.0 / bf)
    vn = s_qn[...] * (1.0 / bf) - mn * mn
    isn = lax.rsqrt(vn + _EPS)
    nsc = ng[...] * isn
    nsh = nbt[...] - mn * nsc
    num_bn = num_ref[...] * nsc + nsh

    mf = s_sf[...] * (1.0 / bf)
    vf = s_qf[...] * (1.0 / bf) - mf * mf
    fsc = fg_e[...] * lax.rsqrt(vf + _EPS)
    fo_e = fe_ref[...] * fsc + (fb_e[...] - mf * fsc)

    mu_nb = nbt[...]
    var_nb = ng[...] * ng[...] * vn / (vn + _EPS)
    m1n = w1[...] * mu_nb
    v1n = w1[...] * w1[...] * var_nb
    sc1n = fg_n[...] * lax.rsqrt(v1n + _EPS)
    fo_n = num_bn * (w1[...] * sc1n) + (fb_n[...] - m1n * sc1n)
    fo[...] = jnp.concatenate([fo_e, fo_n], axis=0)

    be = float(_B * _E)
    m2 = jnp.dot(grt[...], s_ss[...], precision=hi) * (1.0 / be)
    v2 = jnp.dot(grt[...], s_qs[...], precision=hi) * (1.0 / be) - m2 * m2
    sc2 = sg_e[...] * lax.rsqrt(v2 + _EPS)
    sh2 = sb_e[...] - m2 * sc2
    so_e = (se_ref[...] * jnp.dot(ext[...], sc2, precision=hi)
            + jnp.dot(ext[...], sh2, precision=hi))

    e_nb2 = var_nb + mu_nb * mu_nb
    m2n = mu_nb * w2m[...]
    v2n = e_nb2 * w2qm[...] - m2n * m2n
    sc2n = sg_n[...] * lax.rsqrt(v2n + _EPS)
    sh2n = sb_n[...] - m2n * sc2n
    so_n = (jnp.dot(w2selt[...] * jnp.dot(e13t[...], sc2n, precision=hi),
                    num_bn, precision=hi)
            + jnp.dot(e13t[...], sh2n, precision=hi))
    so[...] = jnp.concatenate([so_e, so_n], axis=0)


def _col(r):
    return pl.BlockSpec((r, 1), lambda i: (0, 0))


def _blk(r):
    return pl.BlockSpec((r, _LB), lambda i: (0, i))


def _full(r, c):
    return pl.BlockSpec((r, c), lambda i: (0, 0))


def _gather_call(idx1, idx2, first_flat, second_flat):
    return _build_sc_gather()(idx1, idx2, first_flat, second_flat)


def _f32(shape):
    return jax.ShapeDtypeStruct(shape, jnp.float32)


def kernel(emb_indices, numeric, first_table, second_table, first_num_w,
           second_num_w, numeric_gamma, numeric_beta, first_gamma,
           first_beta, second_gamma, second_beta):
    idx_t = emb_indices.T
    num_t = numeric.T
    ftp = jnp.pad(first_table, ((0, 0), (0, _VP1 - _V)))
    ft128 = jax.lax.optimization_barrier(
        ftp.reshape(_FE * _VP1 // 128, 128))
    first_flat = ft128.reshape(_FE * _VP1)
    spad = jnp.pad(second_table, ((0, 0), (0, _VP2 - _V), (0, 0)))
    st128 = jax.lax.optimization_barrier(
        jnp.swapaxes(spad, 1, 2).reshape(_FE * _E * _VP2 // 128, 128))
    second_flat = st128.reshape(_FE * _E * _VP2)

    idx1 = idx_t + (jnp.arange(_FE, dtype=jnp.int32) * _VP1)[:, None]
    idx2 = (jnp.repeat(idx_t, _E, axis=0)
            + (jnp.arange(_SE, dtype=jnp.int32) * _VP2)[:, None])
    fe_t, se_t = _gather_call(idx1, idx2, first_flat, second_flat)

    sums = pl.pallas_call(
        _reduce_body,
        grid=(_NLB,),
        in_specs=[_blk(_FN), _blk(_FE), _blk(_SE)],
        out_specs=[_col(_FN), _col(_FN), _col(_FE), _col(_FE),
                   _col(_SE), _col(_SE)],
        out_shape=[_f32((_FN, 1)), _f32((_FN, 1)), _f32((_FE, 1)),
                   _f32((_FE, 1)), _f32((_SE, 1)), _f32((_SE, 1))],
    )(num_t, fe_t, se_t)

    c208 = jnp.arange(_SN, dtype=jnp.int32)
    j13 = jnp.arange(_FN, dtype=jnp.int32)
    e13t = (c208[:, None] // _E == j13[None, :]).astype(jnp.float32)
    w2col = second_num_w.reshape(_SN, 1)
    w2selt = e13t * w2col
    c416 = jnp.arange(_SE, dtype=jnp.int32)
    f26 = jnp.arange(_FE, dtype=jnp.int32)
    ext = (c416[:, None] // _E == f26[None, :]).astype(jnp.float32)
    grt = ext.T
    w2m = jnp.mean(second_num_w, axis=1)[:, None]
    w2qm = jnp.mean(second_num_w * second_num_w, axis=1)[:, None]

    params = (
        numeric_gamma[:, None], numeric_beta[:, None], first_num_w[:, None],
        first_gamma[:_FE, None], first_beta[:_FE, None],
        first_gamma[_FE:, None], first_beta[_FE:, None],
        second_gamma[:_FE, None], second_beta[:_FE, None],
        second_gamma[_FE:, None], second_beta[_FE:, None],
        w2m, w2qm, w2selt, ext, grt, e13t,
    )
    param_specs = [
        _col(_FN), _col(_FN), _col(_FN),
        _col(_FE), _col(_FE), _col(_FN), _col(_FN),
        _col(_FE), _col(_FE), _col(_FN), _col(_FN),
        _col(_FN), _col(_FN), _full(_SN, _FN), _full(_SE, _FE),
        _full(_FE, _SE), _full(_SN, _FN),
    ]

    fo, so = pl.pallas_call(
        _norm_body,
        grid=(_NLB,),
        in_specs=[_blk(_FN), _blk(_FE), _blk(_SE)]
                 + [_col(_FN), _col(_FN), _col(_FE), _col(_FE),
                    _col(_SE), _col(_SE)]
                 + param_specs,
        out_specs=[_blk(_FE + _FN), _blk(_SE + _SN)],
        out_shape=[_f32((_FE + _FN, _B)), _f32((_SE + _SN, _B))],
    )(num_t, fe_t, se_t, *sums, *params)

    first_out = fo.T
    second_out = so.reshape(_FE + _FN, _E, _B).transpose(2, 0, 1)
    return first_out, second_out
